# Initial kernel scaffold; baseline (speedup 1.0000x reference)
import jax
import jax.numpy as jnp
from jax.experimental import pallas as pl


def kernel(x, edge_index, W0, W1, W2, b):
    raise NotImplementedError("write your pallas kernel here")


# speedup vs baseline: 15.6283x; 15.6283x over previous
import functools

import jax
import jax.numpy as jnp
from jax import lax
from jax.experimental import pallas as pl
from jax.experimental.pallas import tpu as pltpu
from jax.experimental.pallas import tpu_sc as plsc

N = 10000
E = 320000
D = 128
OUT = 8

NC = 2
NS = 16
NW = NC * NS

CHUNK = 128
NCHUNK = E // CHUNK
NITER = -(-NCHUNK // NW)
NPAD = (N // CHUNK + 1) * CHUNK
RPT = NPAD // NS
TRASH = N


def _mesh():
    return plsc.VectorSubcoreMesh(core_axis_name="c", subcore_axis_name="s")


def _deg_body(ei_hbm, ones_hbm, zeros_hbm, out_hbm, row_v, col_v, rowr_v,
              vals_v, sem, accum):
    cid = lax.axis_index("c")
    sid = lax.axis_index("s")
    wid = sid * NC + cid
    lo = sid * RPT
    pltpu.sync_copy(zeros_hbm.at[pl.ds(lo, RPT)], accum.at[pl.ds(lo, RPT)])
    pltpu.sync_copy(ones_hbm, vals_v)
    plsc.subcore_barrier()

    def step(i, _):
        c = wid + i * NW

        @pl.when(c < NCHUNK)
        def _():
            base = c * CHUNK
            pltpu.sync_copy(ei_hbm.at[0, pl.ds(base, CHUNK)], row_v)
            pltpu.sync_copy(ei_hbm.at[1, pl.ds(base, CHUNK)], col_v)
            for k in range(CHUNK // 16):
                r = row_v[pl.ds(k * 16, 16)]
                cc = col_v[pl.ds(k * 16, 16)]
                rowr_v[pl.ds(k * 16, 16)] = jnp.where(r == cc, TRASH, r)
            pltpu.sync_copy(vals_v, accum.at[rowr_v], add=True)
        return 0

    lax.fori_loop(0, NITER, step, 0)
    plsc.subcore_barrier()
    pltpu.sync_copy(accum.at[pl.ds(lo, RPT)], out_hbm.at[cid, pl.ds(lo, RPT)])


def _msg_body(a_hbm, ei_hbm, zeros_hbm, out_hbm, row_v, col_v, colr_v,
              vals_v, sem, accum):
    cid = lax.axis_index("c")
    sid = lax.axis_index("s")
    wid = sid * NC + cid
    lo = sid * RPT
    pltpu.sync_copy(zeros_hbm.at[pl.ds(lo, RPT)], accum.at[pl.ds(lo, RPT)])
    plsc.subcore_barrier()

    def step(i, _):
        c = wid + i * NW

        @pl.when(c < NCHUNK)
        def _():
            base = c * CHUNK
            pltpu.sync_copy(ei_hbm.at[0, pl.ds(base, CHUNK)], row_v)
            pltpu.sync_copy(ei_hbm.at[1, pl.ds(base, CHUNK)], col_v)
            pltpu.async_copy(a_hbm.at[row_v], vals_v, sem).wait()
            for k in range(CHUNK // 16):
                r = row_v[pl.ds(k * 16, 16)]
                cc = col_v[pl.ds(k * 16, 16)]
                colr_v[pl.ds(k * 16, 16)] = jnp.where(r == cc, TRASH, cc)
            pltpu.sync_copy(vals_v, accum.at[colr_v], add=True)
        return 0

    lax.fori_loop(0, NITER, step, 0)
    plsc.subcore_barrier()
    pltpu.sync_copy(accum.at[pl.ds(lo, RPT)], out_hbm.at[cid, pl.ds(lo, RPT)])


def _sc_scratch():
    return [
        pltpu.VMEM((CHUNK,), jnp.int32),
        pltpu.VMEM((CHUNK,), jnp.int32),
        pltpu.VMEM((CHUNK,), jnp.int32),
        pltpu.VMEM((CHUNK, OUT), jnp.float32),
        pltpu.SemaphoreType.DMA,
        pltpu.VMEM_SHARED((NPAD, OUT), jnp.float32),
    ]


@jax.jit
def _run(x, edge_index, W0, W1, W2, b):
    xp = jnp.pad(x, ((0, NPAD - N), (0, 0)))
    zeros_np8 = jnp.zeros((NPAD, OUT), jnp.float32)
    ones_c8 = jnp.ones((CHUNK, OUT), jnp.float32)
    b2 = b.reshape(1, OUT)

    def tc1(x_ref, w0_ref, w1_ref, w2_ref, y0_ref, y1_ref, y2_ref):
        xv = x_ref[...]
        y0_ref[...] = jnp.dot(xv, w0_ref[...] - w2_ref[...],
                              preferred_element_type=jnp.float32)
        y1_ref[...] = jnp.dot(xv, w1_ref[...],
                              preferred_element_type=jnp.float32)
        y2_ref[...] = jnp.dot(xv, w2_ref[...],
                              preferred_element_type=jnp.float32)

    Y0, Y1, Y2 = pl.pallas_call(
        tc1,
        out_shape=[jax.ShapeDtypeStruct((NPAD, OUT), jnp.float32)] * 3,
    )(xp, W0, W1, W2)

    sc_params = pltpu.CompilerParams(use_tc_tiling_on_sc=False)
    deg_k = pl.kernel(
        _deg_body,
        out_type=jax.ShapeDtypeStruct((NC, NPAD, OUT), jnp.float32),
        mesh=_mesh(),
        scratch_types=_sc_scratch(),
        compiler_params=sc_params,
    )
    Dp = deg_k(edge_index, ones_c8, zeros_np8)

    def tc2(dp_ref, y2_ref, dinv_ref, a2_ref):
        deg = dp_ref[0] + dp_ref[1]
        dinv = jnp.where(deg > 0, lax.rsqrt(jnp.maximum(deg, 1e-12)), 0.0)
        dinv_ref[...] = dinv
        a2_ref[...] = dinv * y2_ref[...]

    dinv8, A2 = pl.pallas_call(
        tc2,
        out_shape=[jax.ShapeDtypeStruct((NPAD, OUT), jnp.float32)] * 2,
    )(Dp, Y2)

    msg_k = pl.kernel(
        _msg_body,
        out_type=jax.ShapeDtypeStruct((NC, NPAD, OUT), jnp.float32),
        mesh=_mesh(),
        scratch_types=_sc_scratch(),
        compiler_params=sc_params,
    )
    Pp = msg_k(A2, edge_index, zeros_np8)

    def tc3(dinv_ref, y1_ref, pp_ref, s_ref):
        dinv = dinv_ref[...]
        p = pp_ref[0] + pp_ref[1]
        s_ref[...] = dinv * y1_ref[...] - 2.0 * dinv * dinv * p

    S = pl.pallas_call(
        tc3,
        out_shape=jax.ShapeDtypeStruct((NPAD, OUT), jnp.float32),
    )(dinv8, Y1, Pp)

    Mp = msg_k(S, edge_index, zeros_np8)

    def tc4(y0_ref, dinv_ref, mp_ref, b_ref, o_ref):
        m = mp_ref[0] + mp_ref[1]
        o_ref[...] = jnp.maximum(
            y0_ref[...] - dinv_ref[...] * m + b_ref[...], 0.0)

    O = pl.pallas_call(
        tc4,
        out_shape=jax.ShapeDtypeStruct((NPAD, OUT), jnp.float32),
    )(Y0, dinv8, Mp, b2)

    return O[:N]


def kernel(x, edge_index, W0, W1, W2, b):
    return _run(x, edge_index, W0, W1, W2, b)


# speedup vs baseline: 42.4028x; 2.7132x over previous
import jax
import jax.numpy as jnp
from jax import lax
from jax.experimental import pallas as pl
from jax.experimental.pallas import tpu as pltpu
from jax.experimental.pallas import tpu_sc as plsc

N = 10000
E = 320000
D = 128
OUT = 8

NC = 2
NS = 16
NW = NC * NS

CHUNK = 128
NCHUNK = E // CHUNK
SB = 10
SEDGE = SB * CHUNK
NSUP = E // SEDGE
NS_ITER = -(-NSUP // NW)
NPAD = (N // CHUNK + 1) * CHUNK
RPT = NPAD // NS
TRASH = N


def _mesh():
    return plsc.VectorSubcoreMesh(core_axis_name="c", subcore_axis_name="s")


def _deg_body(ei3_hbm, ones_hbm, zeros_hbm, dp_hbm, ei2_hbm,
              row_b, col_b, rowr_b, colr_b, ones_v, sem, accum):
    cid = lax.axis_index("c")
    sid = lax.axis_index("s")
    wid = sid * NC + cid
    lo = sid * RPT
    pltpu.sync_copy(zeros_hbm.at[pl.ds(lo, RPT)], accum.at[pl.ds(lo, RPT)])
    pltpu.sync_copy(ones_hbm, ones_v)
    plsc.subcore_barrier()

    def step(i, _):
        sc = wid + i * NW

        @pl.when(sc < NSUP)
        def _():
            pltpu.sync_copy(ei3_hbm.at[0, pl.ds(sc * SB, SB)], row_b)
            pltpu.sync_copy(ei3_hbm.at[1, pl.ds(sc * SB, SB)], col_b)
            for j in range(SB):
                for k in range(CHUNK // 16):
                    sl = pl.ds(k * 16, 16)
                    r = row_b[j, sl]
                    c = col_b[j, sl]
                    m = r == c
                    rowr_b[j, sl] = jnp.where(m, TRASH, r)
                    colr_b[j, sl] = jnp.where(m, TRASH, c)
            descs = [
                pltpu.async_copy(ones_v.at[pl.ds(j * CHUNK, CHUNK)],
                                 accum.at[rowr_b.at[j]], sem, add=True)
                for j in range(SB)
            ]
            pltpu.sync_copy(colr_b, ei2_hbm.at[pl.ds(sc * SB, SB)])
            for d in descs:
                d.wait()
        return 0

    lax.fori_loop(0, NS_ITER, step, 0)
    plsc.subcore_barrier()
    pltpu.sync_copy(accum.at[pl.ds(lo, RPT)], dp_hbm.at[cid, pl.ds(lo, RPT)])


def _msg_body(a_hbm, ei3_hbm, ei2_hbm, zeros_hbm, out_hbm,
              row0, row1, colr0, colr1, vals0, vals1,
              gsem0, gsem1, ssem0, ssem1, accum):
    cid = lax.axis_index("c")
    sid = lax.axis_index("s")
    wid = sid * NC + cid
    lo = sid * RPT
    rows = (row0, row1)
    colrs = (colr0, colr1)
    vals = (vals0, vals1)
    gsems = (gsem0, gsem1)
    ssems = (ssem0, ssem1)

    pltpu.sync_copy(zeros_hbm.at[pl.ds(lo, RPT)], accum.at[pl.ds(lo, RPT)])
    plsc.subcore_barrier()

    def valid(i):
        return (wid + i * NW) < NSUP

    def prefetch(i, b):
        sc = wid + i * NW
        pltpu.sync_copy(ei3_hbm.at[0, pl.ds(sc * SB, SB)], rows[b])
        pltpu.sync_copy(ei2_hbm.at[pl.ds(sc * SB, SB)], colrs[b])
        return [
            pltpu.async_copy(a_hbm.at[rows[b].at[j]],
                             vals[b].at[pl.ds(j * CHUNK, CHUNK)], gsems[b])
            for j in range(SB)
        ]

    g_descs = {}
    s_descs = {}

    @pl.when(valid(0))
    def _():
        g_descs[0] = prefetch(0, 0)

    for i in range(NS_ITER):
        b = i % 2
        nb = (i + 1) % 2
        if i + 1 < NS_ITER:
            @pl.when(valid(i + 1))
            def _(i=i, nb=nb):
                if i - 1 >= 0:
                    for d in s_descs[i - 1]:
                        d.wait()
                g_descs[i + 1] = prefetch(i + 1, nb)

        @pl.when(valid(i))
        def _(i=i, b=b):
            for d in g_descs[i]:
                d.wait()
            s_descs[i] = [
                pltpu.async_copy(vals[b].at[pl.ds(j * CHUNK, CHUNK)],
                                 accum.at[colrs[b].at[j]], ssems[b], add=True)
                for j in range(SB)
            ]

    for i in range(max(NS_ITER - 2, 0), NS_ITER):
        @pl.when(valid(i))
        def _(i=i):
            for d in s_descs[i]:
                d.wait()

    plsc.subcore_barrier()
    pltpu.sync_copy(accum.at[pl.ds(lo, RPT)], out_hbm.at[cid, pl.ds(lo, RPT)])


def _deg_scratch():
    return [
        pltpu.VMEM((SB, CHUNK), jnp.int32),
        pltpu.VMEM((SB, CHUNK), jnp.int32),
        pltpu.VMEM((SB, CHUNK), jnp.int32),
        pltpu.VMEM((SB, CHUNK), jnp.int32),
        pltpu.VMEM((SEDGE, OUT), jnp.float32),
        pltpu.SemaphoreType.DMA,
        pltpu.VMEM_SHARED((NPAD, OUT), jnp.float32),
    ]


def _msg_scratch():
    return [
        pltpu.VMEM((SB, CHUNK), jnp.int32),
        pltpu.VMEM((SB, CHUNK), jnp.int32),
        pltpu.VMEM((SB, CHUNK), jnp.int32),
        pltpu.VMEM((SB, CHUNK), jnp.int32),
        pltpu.VMEM((SEDGE, OUT), jnp.float32),
        pltpu.VMEM((SEDGE, OUT), jnp.float32),
        pltpu.SemaphoreType.DMA,
        pltpu.SemaphoreType.DMA,
        pltpu.SemaphoreType.DMA,
        pltpu.SemaphoreType.DMA,
        pltpu.VMEM_SHARED((NPAD, OUT), jnp.float32),
    ]


@jax.jit
def _run(x, edge_index, W0, W1, W2, b):
    xp = jnp.pad(x, ((0, NPAD - N), (0, 0)))
    ei3 = edge_index.reshape(2, NCHUNK, CHUNK)
    zeros_np8 = jnp.zeros((NPAD, OUT), jnp.float32)
    ones_se8 = jnp.ones((SEDGE, OUT), jnp.float32)
    b2 = b.reshape(1, OUT)

    def tc1(x_ref, w0_ref, w1_ref, w2_ref, y0_ref, y1_ref, y2_ref):
        xv = x_ref[...]
        y0_ref[...] = jnp.dot(xv, w0_ref[...] - w2_ref[...],
                              preferred_element_type=jnp.float32)
        y1_ref[...] = jnp.dot(xv, w1_ref[...],
                              preferred_element_type=jnp.float32)
        y2_ref[...] = jnp.dot(xv, w2_ref[...],
                              preferred_element_type=jnp.float32)

    Y0, Y1, Y2 = pl.pallas_call(
        tc1,
        out_shape=[jax.ShapeDtypeStruct((NPAD, OUT), jnp.float32)] * 3,
    )(xp, W0, W1, W2)

    sc_params = pltpu.CompilerParams(use_tc_tiling_on_sc=False)

    deg_k = pl.kernel(
        _deg_body,
        out_type=[
            jax.ShapeDtypeStruct((NC, NPAD, OUT), jnp.float32),
            jax.ShapeDtypeStruct((NCHUNK, CHUNK), jnp.int32),
        ],
        mesh=_mesh(),
        scratch_types=_deg_scratch(),
        compiler_params=sc_params,
    )
    Dp, ei2 = deg_k(ei3, ones_se8, zeros_np8)

    def tc2(dp_ref, y2_ref, dinv_ref, a2_ref):
        deg = dp_ref[0] + dp_ref[1]
        dinv = jnp.where(deg > 0, lax.rsqrt(jnp.maximum(deg, 1e-12)), 0.0)
        dinv_ref[...] = dinv
        a2_ref[...] = dinv * y2_ref[...]

    dinv8, A2 = pl.pallas_call(
        tc2,
        out_shape=[jax.ShapeDtypeStruct((NPAD, OUT), jnp.float32)] * 2,
    )(Dp, Y2)

    msg_k = pl.kernel(
        _msg_body,
        out_type=jax.ShapeDtypeStruct((NC, NPAD, OUT), jnp.float32),
        mesh=_mesh(),
        scratch_types=_msg_scratch(),
        compiler_params=sc_params,
    )
    Pp = msg_k(A2, ei3, ei2, zeros_np8)

    def tc3(dinv_ref, y1_ref, pp_ref, s_ref):
        dinv = dinv_ref[...]
        p = pp_ref[0] + pp_ref[1]
        s_ref[...] = dinv * y1_ref[...] - 2.0 * dinv * dinv * p

    S = pl.pallas_call(
        tc3,
        out_shape=jax.ShapeDtypeStruct((NPAD, OUT), jnp.float32),
    )(dinv8, Y1, Pp)

    Mp = msg_k(S, ei3, ei2, zeros_np8)

    def tc4(y0_ref, dinv_ref, mp_ref, b_ref, o_ref):
        m = mp_ref[0] + mp_ref[1]
        o_ref[...] = jnp.maximum(
            y0_ref[...] - dinv_ref[...] * m + b_ref[...], 0.0)

    O = pl.pallas_call(
        tc4,
        out_shape=jax.ShapeDtypeStruct((NPAD, OUT), jnp.float32),
    )(Y0, dinv8, Mp, b2)

    return O[:N]


def kernel(x, edge_index, W0, W1, W2, b):
    return _run(x, edge_index, W0, W1, W2, b)


# speedup vs baseline: 43.3881x; 1.0232x over previous
import jax
import jax.numpy as jnp
from jax import lax
from jax.experimental import pallas as pl
from jax.experimental.pallas import tpu as pltpu
from jax.experimental.pallas import tpu_sc as plsc

N = 10000
E = 320000
D = 128
OUT = 8

NC = 2
NS = 16
NW = NC * NS

CHUNK = 128
NCHUNK = E // CHUNK
SB = 20
SEDGE = SB * CHUNK
NSUP = E // SEDGE
NS_ITER = -(-NSUP // NW)
NPAD = (N // CHUNK + 1) * CHUNK
RPT = NPAD // NS
TRASH = N


def _mesh():
    return plsc.VectorSubcoreMesh(core_axis_name="c", subcore_axis_name="s")


def _deg_body(ei3_hbm, ones_hbm, zeros_hbm, dp_hbm, ei2_hbm,
              row_b, col_b, rowr_b, colr_b, ones_v, sem, accum):
    cid = lax.axis_index("c")
    sid = lax.axis_index("s")
    wid = sid * NC + cid
    lo = sid * RPT
    pltpu.sync_copy(zeros_hbm.at[pl.ds(lo, RPT)], accum.at[pl.ds(lo, RPT)])
    pltpu.sync_copy(ones_hbm, ones_v)
    plsc.subcore_barrier()

    def step(i, _):
        sc = wid + i * NW

        @pl.when(sc < NSUP)
        def _():
            pltpu.sync_copy(ei3_hbm.at[0, pl.ds(sc * SB, SB)], row_b)
            pltpu.sync_copy(ei3_hbm.at[1, pl.ds(sc * SB, SB)], col_b)
            for j in range(SB):
                for k in range(CHUNK // 16):
                    sl = pl.ds(k * 16, 16)
                    r = row_b[j, sl]
                    c = col_b[j, sl]
                    m = r == c
                    rowr_b[j, sl] = jnp.where(m, TRASH, r)
                    colr_b[j, sl] = jnp.where(m, TRASH, c)
            descs = [
                pltpu.async_copy(ones_v.at[pl.ds(j * CHUNK, CHUNK)],
                                 accum.at[rowr_b.at[j]], sem, add=True)
                for j in range(SB)
            ]
            pltpu.sync_copy(colr_b, ei2_hbm.at[pl.ds(sc * SB, SB)])
            for d in descs:
                d.wait()
        return 0

    lax.fori_loop(0, NS_ITER, step, 0)
    plsc.subcore_barrier()
    pltpu.sync_copy(accum.at[pl.ds(lo, RPT)], dp_hbm.at[cid, pl.ds(lo, RPT)])


def _msg_body(a_hbm, ei3_hbm, ei2_hbm, zeros_hbm, out_hbm,
              row0, row1, colr0, colr1, vals0, vals1,
              gsem0, gsem1, ssem0, ssem1, accum):
    cid = lax.axis_index("c")
    sid = lax.axis_index("s")
    wid = sid * NC + cid
    lo = sid * RPT
    rows = (row0, row1)
    colrs = (colr0, colr1)
    vals = (vals0, vals1)
    gsems = (gsem0, gsem1)
    ssems = (ssem0, ssem1)

    pltpu.sync_copy(zeros_hbm.at[pl.ds(lo, RPT)], accum.at[pl.ds(lo, RPT)])
    plsc.subcore_barrier()

    def valid(i):
        return (wid + i * NW) < NSUP

    def prefetch(i, b):
        sc = wid + i * NW
        pltpu.sync_copy(ei3_hbm.at[0, pl.ds(sc * SB, SB)], rows[b])
        pltpu.sync_copy(ei2_hbm.at[pl.ds(sc * SB, SB)], colrs[b])
        return [
            pltpu.async_copy(a_hbm.at[rows[b].at[j]],
                             vals[b].at[pl.ds(j * CHUNK, CHUNK)], gsems[b])
            for j in range(SB)
        ]

    g_descs = {}
    s_descs = {}

    @pl.when(valid(0))
    def _():
        g_descs[0] = prefetch(0, 0)

    for i in range(NS_ITER):
        b = i % 2
        nb = (i + 1) % 2
        if i + 1 < NS_ITER:
            @pl.when(valid(i + 1))
            def _(i=i, nb=nb):
                if i - 1 >= 0:
                    for d in s_descs[i - 1]:
                        d.wait()
                g_descs[i + 1] = prefetch(i + 1, nb)

        @pl.when(valid(i))
        def _(i=i, b=b):
            for d in g_descs[i]:
                d.wait()
            s_descs[i] = [
                pltpu.async_copy(vals[b].at[pl.ds(j * CHUNK, CHUNK)],
                                 accum.at[colrs[b].at[j]], ssems[b], add=True)
                for j in range(SB)
            ]

    for i in range(max(NS_ITER - 2, 0), NS_ITER):
        @pl.when(valid(i))
        def _(i=i):
            for d in s_descs[i]:
                d.wait()

    plsc.subcore_barrier()
    pltpu.sync_copy(accum.at[pl.ds(lo, RPT)], out_hbm.at[cid, pl.ds(lo, RPT)])


def _deg_scratch():
    return [
        pltpu.VMEM((SB, CHUNK), jnp.int32),
        pltpu.VMEM((SB, CHUNK), jnp.int32),
        pltpu.VMEM((SB, CHUNK), jnp.int32),
        pltpu.VMEM((SB, CHUNK), jnp.int32),
        pltpu.VMEM((SEDGE, OUT), jnp.float32),
        pltpu.SemaphoreType.DMA,
        pltpu.VMEM_SHARED((NPAD, OUT), jnp.float32),
    ]


def _msg_scratch():
    return [
        pltpu.VMEM((SB, CHUNK), jnp.int32),
        pltpu.VMEM((SB, CHUNK), jnp.int32),
        pltpu.VMEM((SB, CHUNK), jnp.int32),
        pltpu.VMEM((SB, CHUNK), jnp.int32),
        pltpu.VMEM((SEDGE, OUT), jnp.float32),
        pltpu.VMEM((SEDGE, OUT), jnp.float32),
        pltpu.SemaphoreType.DMA,
        pltpu.SemaphoreType.DMA,
        pltpu.SemaphoreType.DMA,
        pltpu.SemaphoreType.DMA,
        pltpu.VMEM_SHARED((NPAD, OUT), jnp.float32),
    ]


@jax.jit
def _run(x, edge_index, W0, W1, W2, b):
    xp = jnp.pad(x, ((0, NPAD - N), (0, 0)))
    ei3 = edge_index.reshape(2, NCHUNK, CHUNK)
    zeros_np8 = jnp.zeros((NPAD, OUT), jnp.float32)
    ones_se8 = jnp.ones((SEDGE, OUT), jnp.float32)
    b2 = b.reshape(1, OUT)

    sc_params = pltpu.CompilerParams(use_tc_tiling_on_sc=False)

    deg_k = pl.kernel(
        _deg_body,
        out_type=[
            jax.ShapeDtypeStruct((NC, NPAD, OUT), jnp.float32),
            jax.ShapeDtypeStruct((NCHUNK, CHUNK), jnp.int32),
        ],
        mesh=_mesh(),
        scratch_types=_deg_scratch(),
        compiler_params=sc_params,
    )
    Dp, ei2 = deg_k(ei3, ones_se8, zeros_np8)

    def tc2(dp_ref, x_ref, w0_ref, w1_ref, w2_ref,
            dinv_ref, a2_ref, y0_ref, y1_ref):
        deg = dp_ref[0] + dp_ref[1]
        dinv = jnp.where(deg > 0, lax.rsqrt(jnp.maximum(deg, 1e-12)), 0.0)
        dinv_ref[...] = dinv
        xv = x_ref[...]
        a2_ref[...] = dinv * jnp.dot(xv, w2_ref[...],
                                     preferred_element_type=jnp.float32)
        y0_ref[...] = jnp.dot(xv, w0_ref[...] - w2_ref[...],
                              preferred_element_type=jnp.float32)
        y1_ref[...] = jnp.dot(xv, w1_ref[...],
                              preferred_element_type=jnp.float32)

    dinv8, A2, Y0, Y1 = pl.pallas_call(
        tc2,
        out_shape=[jax.ShapeDtypeStruct((NPAD, OUT), jnp.float32)] * 4,
    )(Dp, xp, W0, W1, W2)

    msg_k = pl.kernel(
        _msg_body,
        out_type=jax.ShapeDtypeStruct((NC, NPAD, OUT), jnp.float32),
        mesh=_mesh(),
        scratch_types=_msg_scratch(),
        compiler_params=sc_params,
    )
    Pp = msg_k(A2, ei3, ei2, zeros_np8)

    def tc3(dinv_ref, y1_ref, pp_ref, s_ref):
        dinv = dinv_ref[...]
        p = pp_ref[0] + pp_ref[1]
        s_ref[...] = dinv * y1_ref[...] - 2.0 * dinv * dinv * p

    S = pl.pallas_call(
        tc3,
        out_shape=jax.ShapeDtypeStruct((NPAD, OUT), jnp.float32),
    )(dinv8, Y1, Pp)

    Mp = msg_k(S, ei3, ei2, zeros_np8)

    def tc4(y0_ref, dinv_ref, mp_ref, b_ref, o_ref):
        m = mp_ref[0] + mp_ref[1]
        o_ref[...] = jnp.maximum(
            y0_ref[...] - dinv_ref[...] * m + b_ref[...], 0.0)

    O = pl.pallas_call(
        tc4,
        out_shape=jax.ShapeDtypeStruct((NPAD, OUT), jnp.float32),
    )(Y0, dinv8, Mp, b2)

    return O[:N]


def kernel(x, edge_index, W0, W1, W2, b):
    return _run(x, edge_index, W0, W1, W2, b)
